# Initial kernel scaffold; baseline (speedup 1.0000x reference)
import jax
import jax.numpy as jnp
from jax.experimental import pallas as pl


def kernel(node_features, edge_index, W1, b1, W2, b2):
    raise NotImplementedError("write your pallas kernel here")


# speedup vs baseline: 2.6877x; 2.6877x over previous
import functools

import jax
import jax.numpy as jnp
from jax import lax
from jax.experimental import pallas as pl
from jax.experimental.pallas import tpu as pltpu
from jax.experimental.pallas import tpu_sc as plsc

N = 10000
NPAD = 10240
E = 320000
EPAD = 327680
D = 128
NC = 2
NS = 16
ROWS_PER_TILE = NPAD // NS
ECHUNK = 128
EITERS = (EPAD // (NC * NS)) // ECHUNK
DCHUNK = 80
DITERS = (E // NS) // DCHUNK
PAD_NODE = 10200
BLK = 1024
GRID = NPAD // BLK

_mesh = plsc.VectorSubcoreMesh(core_axis_name="c", subcore_axis_name="s")


def _rsqrt16(d):
    i = lax.bitcast_convert_type(d, jnp.int32)
    i = jnp.int32(0x5F3759DF) - lax.shift_right_arithmetic(i, 1)
    y = lax.bitcast_convert_type(i, jnp.float32)
    for _ in range(3):
        y = y * (jnp.float32(1.5) - jnp.float32(0.5) * d * y * y)
    return jnp.where(d > jnp.float32(0.5), y, jnp.float32(0.0))


@functools.partial(
    pl.kernel,
    out_type=(
        jax.ShapeDtypeStruct((NPAD,), jnp.float32),
        jax.ShapeDtypeStruct((NPAD,), jnp.float32),
    ),
    mesh=_mesh,
    scratch_types=[
        pltpu.VMEM_SHARED((NPAD,), jnp.float32),
        pltpu.VMEM((DCHUNK,), jnp.int32),
        pltpu.VMEM((DCHUNK,), jnp.float32),
        pltpu.VMEM((ROWS_PER_TILE,), jnp.float32),
    ],
)
def _deg_norms(src_hbm, dst_hbm, z1_hbm, ns_hbm, nd_hbm, hist, idxb, onesb, normv):
    c = lax.axis_index("c")
    s = lax.axis_index("s")
    r0 = pl.multiple_of(s * ROWS_PER_TILE, 8)
    pltpu.sync_copy(z1_hbm.at[pl.ds(r0, ROWS_PER_TILE)], hist.at[pl.ds(r0, ROWS_PER_TILE)])
    for j in range(DCHUNK // 16):
        onesb[pl.ds(j * 16, 16)] = jnp.full((16,), 1.0, jnp.float32)
    plsc.subcore_barrier()

    def _count(ref):
        def body(i, carry):
            base = pl.multiple_of(s * (E // NS) + i * DCHUNK, 8)
            pltpu.sync_copy(ref.at[pl.ds(base, DCHUNK)], idxb)
            pltpu.sync_copy(onesb, hist.at[idxb], add=True)
            return carry
        lax.fori_loop(0, DITERS, body, 0)

    @pl.when(c == 0)
    def _():
        _count(src_hbm)

    @pl.when(c == 1)
    def _():
        _count(dst_hbm)

    plsc.subcore_barrier()
    pltpu.sync_copy(hist.at[pl.ds(r0, ROWS_PER_TILE)], normv)
    for j in range(ROWS_PER_TILE // 16):
        normv[pl.ds(j * 16, 16)] = _rsqrt16(normv[pl.ds(j * 16, 16)])

    @pl.when(c == 0)
    def _():
        pltpu.sync_copy(normv, ns_hbm.at[pl.ds(r0, ROWS_PER_TILE)])

    @pl.when(c == 1)
    def _():
        pltpu.sync_copy(normv, nd_hbm.at[pl.ds(r0, ROWS_PER_TILE)])


@functools.partial(
    pl.kernel,
    out_type=jax.ShapeDtypeStruct((NC, NPAD, D), jnp.float32),
    mesh=_mesh,
    scratch_types=[
        pltpu.VMEM_SHARED((NPAD, D), jnp.float32),
        pltpu.VMEM((ECHUNK,), jnp.int32),
        pltpu.VMEM((ECHUNK,), jnp.int32),
        pltpu.VMEM((ECHUNK, D), jnp.float32),
        pltpu.SemaphoreType.DMA,
    ],
)
def _edge_scatter(h_hbm, src_hbm, dst_hbm, z2_hbm, out_hbm, acc, sidx, didx, rows, sem):
    c = lax.axis_index("c")
    s = lax.axis_index("s")
    w = c * NS + s
    r0 = pl.multiple_of(s * ROWS_PER_TILE, 8)
    pltpu.sync_copy(z2_hbm.at[pl.ds(r0, ROWS_PER_TILE)], acc.at[pl.ds(r0, ROWS_PER_TILE)])
    plsc.subcore_barrier()

    def body(i, carry):
        base = pl.multiple_of(w * (EPAD // (NC * NS)) + i * ECHUNK, 8)
        pltpu.sync_copy(src_hbm.at[pl.ds(base, ECHUNK)], sidx)
        pltpu.sync_copy(dst_hbm.at[pl.ds(base, ECHUNK)], didx)
        pltpu.async_copy(h_hbm.at[sidx], rows, sem).wait()
        pltpu.sync_copy(rows, acc.at[didx], add=True)
        return carry

    lax.fori_loop(0, EITERS, body, 0)
    plsc.subcore_barrier()
    pltpu.sync_copy(acc.at[pl.ds(r0, ROWS_PER_TILE)], out_hbm.at[c, pl.ds(r0, ROWS_PER_TILE)])


def _dot(a, b):
    return lax.dot_general(a, b, (((1,), (0,)), ((), ())),
                           preferred_element_type=jnp.float32,
                           precision=lax.Precision.HIGHEST)


def _mm_body(x_ref, ns_ref, w_ref, o_ref):
    o_ref[...] = _dot(x_ref[...] * ns_ref[...], w_ref[...])


def _mm_call(x, ns2, W):
    return pl.pallas_call(
        _mm_body,
        grid=(GRID,),
        in_specs=[
            pl.BlockSpec((BLK, D), lambda i: (i, 0)),
            pl.BlockSpec((BLK, 1), lambda i: (i, 0)),
            pl.BlockSpec((D, D), lambda i: (0, 0)),
        ],
        out_specs=pl.BlockSpec((BLK, D), lambda i: (i, 0)),
        out_shape=jax.ShapeDtypeStruct((NPAD, D), jnp.float32),
    )(x, ns2, W)


def _mid_body(a_ref, nd_ref, ns_ref, b_ref, w_ref, o_ref):
    agg = a_ref[0] + a_ref[1]
    mid = jnp.maximum(agg * nd_ref[...] + b_ref[...], 0.0)
    o_ref[...] = _dot(mid * ns_ref[...], w_ref[...])


def _mid_call(aggp, nd2, ns2, b1r, W2):
    return pl.pallas_call(
        _mid_body,
        grid=(GRID,),
        in_specs=[
            pl.BlockSpec((NC, BLK, D), lambda i: (0, i, 0)),
            pl.BlockSpec((BLK, 1), lambda i: (i, 0)),
            pl.BlockSpec((BLK, 1), lambda i: (i, 0)),
            pl.BlockSpec((1, D), lambda i: (0, 0)),
            pl.BlockSpec((D, D), lambda i: (0, 0)),
        ],
        out_specs=pl.BlockSpec((BLK, D), lambda i: (i, 0)),
        out_shape=jax.ShapeDtypeStruct((NPAD, D), jnp.float32),
    )(aggp, nd2, ns2, b1r, W2)


def _fin_body(a_ref, nd_ref, b_ref, o_ref):
    agg = a_ref[0] + a_ref[1]
    o_ref[...] = jnp.maximum(agg * nd_ref[...] + b_ref[...], 0.0)


def _fin_call(aggp, nd2, b2r):
    return pl.pallas_call(
        _fin_body,
        grid=(GRID,),
        in_specs=[
            pl.BlockSpec((NC, BLK, D), lambda i: (0, i, 0)),
            pl.BlockSpec((BLK, 1), lambda i: (i, 0)),
            pl.BlockSpec((1, D), lambda i: (0, 0)),
        ],
        out_specs=pl.BlockSpec((BLK, D), lambda i: (i, 0)),
        out_shape=jax.ShapeDtypeStruct((NPAD, D), jnp.float32),
    )(aggp, nd2, b2r)


def kernel(node_features, edge_index, W1, b1, W2, b2):
    src = edge_index[0].astype(jnp.int32)
    dst = edge_index[1].astype(jnp.int32)
    pad = jnp.full((EPAD - E,), PAD_NODE, jnp.int32)
    src_p = jnp.concatenate([src, pad])
    dst_p = jnp.concatenate([dst, pad])
    x_p = jnp.pad(node_features, ((0, NPAD - N), (0, 0)))
    z1 = jnp.zeros((NPAD,), jnp.float32)
    z2 = jnp.zeros((NPAD, D), jnp.float32)

    ns, nd = _deg_norms(src, dst, z1)
    ns2 = ns[:, None]
    nd2 = nd[:, None]

    h1 = _mm_call(x_p, ns2, W1)
    agg1 = _edge_scatter(h1, src_p, dst_p, z2)
    h2 = _mid_call(agg1, nd2, ns2, b1[None, :], W2)
    agg2 = _edge_scatter(h2, src_p, dst_p, z2)
    out = _fin_call(agg2, nd2, b2[None, :])
    return out[:N]


# speedup vs baseline: 4.1197x; 1.5328x over previous
import functools

import jax
import jax.numpy as jnp
from jax import lax
from jax.experimental import pallas as pl
from jax.experimental.pallas import tpu as pltpu
from jax.experimental.pallas import tpu_sc as plsc

N = 10000
NPAD = 10240
E = 320000
EPAD = 327680
D = 128
NC = 2
NS = 16
ROWS_PER_TILE = NPAD // NS
ECHUNK = 64
EITERS = (EPAD // (NC * NS)) // ECHUNK
DCHUNK = 64
DITERS = (EPAD // NS) // DCHUNK
NBUF = 4
EGROUPS = EITERS // NBUF
PAD_NODE = 10200
BLK = 1024
GRID = NPAD // BLK

_mesh = plsc.VectorSubcoreMesh(core_axis_name="c", subcore_axis_name="s")


def _rsqrt16(d):
    i = lax.bitcast_convert_type(d, jnp.int32)
    i = jnp.int32(0x5F3759DF) - lax.shift_right_arithmetic(i, 1)
    y = lax.bitcast_convert_type(i, jnp.float32)
    for _ in range(3):
        y = y * (jnp.float32(1.5) - jnp.float32(0.5) * d * y * y)
    return jnp.where(d > jnp.float32(0.5), y, jnp.float32(0.0))


@functools.partial(
    pl.kernel,
    out_type=(
        jax.ShapeDtypeStruct((NPAD,), jnp.float32),
        jax.ShapeDtypeStruct((NPAD,), jnp.float32),
    ),
    mesh=_mesh,
    scratch_types=[
        pltpu.VMEM_SHARED((NPAD,), jnp.float32),
        pltpu.VMEM((DITERS, DCHUNK), jnp.int32),
        pltpu.VMEM((DCHUNK,), jnp.float32),
        pltpu.VMEM((ROWS_PER_TILE,), jnp.float32),
    ]
    + [pltpu.SemaphoreType.DMA] * NBUF,
)
def _deg_norms(src3_hbm, dst3_hbm, z1_hbm, ns_hbm, nd_hbm, hist, idxa, onesb, normv, *sems):
    c = lax.axis_index("c")
    s = lax.axis_index("s")
    r0 = pl.multiple_of(s * ROWS_PER_TILE, 8)
    pltpu.sync_copy(z1_hbm.at[pl.ds(r0, ROWS_PER_TILE)], hist.at[pl.ds(r0, ROWS_PER_TILE)])
    for j in range(DCHUNK // 16):
        onesb[pl.ds(j * 16, 16)] = jnp.full((16,), 1.0, jnp.float32)

    @pl.when(c == 0)
    def _():
        pltpu.sync_copy(src3_hbm.at[2 * s], idxa.at[pl.ds(0, DITERS // 2)])
        pltpu.sync_copy(src3_hbm.at[2 * s + 1], idxa.at[pl.ds(DITERS // 2, DITERS // 2)])

    @pl.when(c == 1)
    def _():
        pltpu.sync_copy(dst3_hbm.at[2 * s], idxa.at[pl.ds(0, DITERS // 2)])
        pltpu.sync_copy(dst3_hbm.at[2 * s + 1], idxa.at[pl.ds(DITERS // 2, DITERS // 2)])

    plsc.subcore_barrier()

    def _start(b, ci):
        pltpu.async_copy(onesb, hist.at[idxa.at[ci]], sems[b], add=True)

    def _wait(b):
        pltpu.make_async_copy(onesb, hist.at[idxa.at[0]], sems[b]).wait()

    for b in range(NBUF):
        _start(b, b)

    def body(g, carry):
        for b in range(NBUF):
            _wait(b)
            _start(b, (g + 1) * NBUF + b)
        return carry

    lax.fori_loop(0, DITERS // NBUF - 1, body, 0)
    for b in range(NBUF):
        _wait(b)

    plsc.subcore_barrier()
    pltpu.sync_copy(hist.at[pl.ds(r0, ROWS_PER_TILE)], normv)
    for j in range(ROWS_PER_TILE // 16):
        normv[pl.ds(j * 16, 16)] = _rsqrt16(normv[pl.ds(j * 16, 16)])

    @pl.when(c == 0)
    def _():
        pltpu.sync_copy(normv, ns_hbm.at[pl.ds(r0, ROWS_PER_TILE)])

    @pl.when(c == 1)
    def _():
        pltpu.sync_copy(normv, nd_hbm.at[pl.ds(r0, ROWS_PER_TILE)])


@functools.partial(
    pl.kernel,
    out_type=jax.ShapeDtypeStruct((NC, NPAD, D), jnp.float32),
    mesh=_mesh,
    scratch_types=[
        pltpu.VMEM_SHARED((NPAD, D), jnp.float32),
        pltpu.VMEM((2 * NBUF, ECHUNK), jnp.int32),
        pltpu.VMEM((2 * NBUF, ECHUNK), jnp.int32),
    ]
    + [pltpu.VMEM((ECHUNK, D), jnp.float32)] * NBUF
    + [pltpu.SemaphoreType.DMA] * (2 * NBUF + 2),
)
def _edge_scatter(h_hbm, src3_hbm, dst3_hbm, z2_hbm, out_hbm, acc, sidxg, didxg, *bufs):
    rows = bufs[:NBUF]
    gsem = bufs[NBUF:2 * NBUF]
    ssem = bufs[2 * NBUF:3 * NBUF]
    issem, idsem = bufs[3 * NBUF:]
    c = lax.axis_index("c")
    s = lax.axis_index("s")
    w = c * NS + s
    r0 = pl.multiple_of(s * ROWS_PER_TILE, 8)
    pltpu.sync_copy(z2_hbm.at[pl.ds(r0, ROWS_PER_TILE)], acc.at[pl.ds(r0, ROWS_PER_TILE)])
    pltpu.sync_copy(src3_hbm.at[w, pl.ds(0, NBUF)], sidxg.at[pl.ds(0, NBUF)])
    pltpu.sync_copy(dst3_hbm.at[w, pl.ds(0, NBUF)], didxg.at[pl.ds(0, NBUF)])
    plsc.subcore_barrier()

    def _istart(g, slot):
        pltpu.async_copy(src3_hbm.at[w, pl.ds(g * NBUF, NBUF)],
                         sidxg.at[pl.ds(slot * NBUF, NBUF)], issem)
        pltpu.async_copy(dst3_hbm.at[w, pl.ds(g * NBUF, NBUF)],
                         didxg.at[pl.ds(slot * NBUF, NBUF)], idsem)

    def _iwait():
        pltpu.make_async_copy(src3_hbm.at[w, pl.ds(0, NBUF)],
                              sidxg.at[pl.ds(0, NBUF)], issem).wait()
        pltpu.make_async_copy(dst3_hbm.at[w, pl.ds(0, NBUF)],
                              didxg.at[pl.ds(0, NBUF)], idsem).wait()

    def _gstart(b, slot):
        pltpu.async_copy(h_hbm.at[sidxg.at[slot * NBUF + b]], rows[b], gsem[b])

    def _gwait(b):
        pltpu.make_async_copy(h_hbm.at[sidxg.at[0]], rows[b], gsem[b]).wait()

    def _sstart(b, slot):
        pltpu.async_copy(rows[b], acc.at[didxg.at[slot * NBUF + b]], ssem[b], add=True)

    def _swait(b):
        pltpu.make_async_copy(rows[b], acc.at[didxg.at[0]], ssem[b]).wait()

    for b in range(NBUF):
        _gstart(b, 0)

    def body(g, carry):
        p = lax.rem(g, 2)
        _istart(g + 1, 1 - p)
        for b in range(NBUF):
            _gwait(b)
            _sstart(b, p)
        _iwait()
        for b in range(NBUF):
            _swait(b)
            _gstart(b, 1 - p)
        return carry

    lax.fori_loop(0, EGROUPS - 1, body, 0)
    pl_last = (EGROUPS - 1) % 2
    for b in range(NBUF):
        _gwait(b)
        _sstart(b, pl_last)
    for b in range(NBUF):
        _swait(b)
    plsc.subcore_barrier()
    pltpu.sync_copy(acc.at[pl.ds(r0, ROWS_PER_TILE)], out_hbm.at[c, pl.ds(r0, ROWS_PER_TILE)])


def _dot(a, b):
    return lax.dot_general(a, b, (((1,), (0,)), ((), ())),
                           preferred_element_type=jnp.float32,
                           precision=lax.Precision.HIGHEST)


def _mm_body(x_ref, ns_ref, w_ref, o_ref):
    o_ref[...] = _dot(x_ref[...] * ns_ref[...], w_ref[...])


def _mm_call(x, ns2, W):
    return pl.pallas_call(
        _mm_body,
        grid=(GRID,),
        in_specs=[
            pl.BlockSpec((BLK, D), lambda i: (i, 0)),
            pl.BlockSpec((BLK, 1), lambda i: (i, 0)),
            pl.BlockSpec((D, D), lambda i: (0, 0)),
        ],
        out_specs=pl.BlockSpec((BLK, D), lambda i: (i, 0)),
        out_shape=jax.ShapeDtypeStruct((NPAD, D), jnp.float32),
    )(x, ns2, W)


def _mid_body(a_ref, nd_ref, ns_ref, b_ref, w_ref, o_ref):
    agg = a_ref[0] + a_ref[1]
    mid = jnp.maximum(agg * nd_ref[...] + b_ref[...], 0.0)
    o_ref[...] = _dot(mid * ns_ref[...], w_ref[...])


def _mid_call(aggp, nd2, ns2, b1r, W2):
    return pl.pallas_call(
        _mid_body,
        grid=(GRID,),
        in_specs=[
            pl.BlockSpec((NC, BLK, D), lambda i: (0, i, 0)),
            pl.BlockSpec((BLK, 1), lambda i: (i, 0)),
            pl.BlockSpec((BLK, 1), lambda i: (i, 0)),
            pl.BlockSpec((1, D), lambda i: (0, 0)),
            pl.BlockSpec((D, D), lambda i: (0, 0)),
        ],
        out_specs=pl.BlockSpec((BLK, D), lambda i: (i, 0)),
        out_shape=jax.ShapeDtypeStruct((NPAD, D), jnp.float32),
    )(aggp, nd2, ns2, b1r, W2)


def _fin_body(a_ref, nd_ref, b_ref, o_ref):
    agg = a_ref[0] + a_ref[1]
    o_ref[...] = jnp.maximum(agg * nd_ref[...] + b_ref[...], 0.0)


def _fin_call(aggp, nd2, b2r):
    return pl.pallas_call(
        _fin_body,
        grid=(GRID,),
        in_specs=[
            pl.BlockSpec((NC, BLK, D), lambda i: (0, i, 0)),
            pl.BlockSpec((BLK, 1), lambda i: (i, 0)),
            pl.BlockSpec((1, D), lambda i: (0, 0)),
        ],
        out_specs=pl.BlockSpec((BLK, D), lambda i: (i, 0)),
        out_shape=jax.ShapeDtypeStruct((NPAD, D), jnp.float32),
    )(aggp, nd2, b2r)


def kernel(node_features, edge_index, W1, b1, W2, b2):
    src = edge_index[0].astype(jnp.int32)
    dst = edge_index[1].astype(jnp.int32)
    pad = jnp.full((EPAD - E,), PAD_NODE, jnp.int32)
    src_p = jnp.concatenate([src, pad]).reshape(NC * NS, EITERS, ECHUNK)
    dst_p = jnp.concatenate([dst, pad]).reshape(NC * NS, EITERS, ECHUNK)
    x_p = jnp.pad(node_features, ((0, NPAD - N), (0, 0)))
    z1 = jnp.zeros((NPAD,), jnp.float32)
    z2 = jnp.zeros((NPAD, D), jnp.float32)

    ns, nd = _deg_norms(src_p, dst_p, z1)
    ns2 = ns[:, None]
    nd2 = nd[:, None]

    h1 = _mm_call(x_p, ns2, W1)
    agg1 = _edge_scatter(h1, src_p, dst_p, z2)
    h2 = _mid_call(agg1, nd2, ns2, b1[None, :], W2)
    agg2 = _edge_scatter(h2, src_p, dst_p, z2)
    out = _fin_call(agg2, nd2, b2[None, :])
    return out[:N]


# speedup vs baseline: 4.6272x; 1.1232x over previous
import functools

import jax
import jax.numpy as jnp
from jax import lax
from jax.experimental import pallas as pl
from jax.experimental.pallas import tpu as pltpu
from jax.experimental.pallas import tpu_sc as plsc

N = 10000
NPH = 10240
HHALF = NPH // 2
NPA = 10112
NHIST = 10240
E = 320000
D = 128
NC = 2
NS = 16
NT = NC * NS
EPT = 10240
EPAD = NT * EPT
ACC_RPT = NPA // NS
HROWS = HHALF // NS
HR = NHIST // NS
PAD_NODE = 10016
ECHUNK = 32
NBUF = 2
LCH = EPT // ECHUNK
LGRP = LCH // NBUF
DCHUNK = 64
DITERS = EPAD // NS // DCHUNK
DBUF = 4
PCH = EPT // DCHUNK
BLKH = 640
BLKA = 632
GRID = 16

_mesh = plsc.VectorSubcoreMesh(core_axis_name="c", subcore_axis_name="s")


def _rsqrt16(d):
    i = lax.bitcast_convert_type(d, jnp.int32)
    i = jnp.int32(0x5F3759DF) - lax.shift_right_arithmetic(i, 1)
    y = lax.bitcast_convert_type(i, jnp.float32)
    for _ in range(3):
        y = y * (jnp.float32(1.5) - jnp.float32(0.5) * d * y * y)
    return jnp.where(d > jnp.float32(0.5), y, jnp.float32(0.0))


@functools.partial(
    pl.kernel,
    out_type=(
        jax.ShapeDtypeStruct((NHIST,), jnp.float32),
        jax.ShapeDtypeStruct((NHIST,), jnp.float32),
        jax.ShapeDtypeStruct((NC * NT * 2 * EPT,), jnp.int32),
    ),
    mesh=_mesh,
    scratch_types=[
        pltpu.VMEM_SHARED((NHIST,), jnp.float32),
        pltpu.VMEM((DITERS, DCHUNK), jnp.int32),
        pltpu.VMEM((PCH, DCHUNK), jnp.int32),
        pltpu.VMEM((PCH, DCHUNK), jnp.int32),
        pltpu.VMEM((EPT,), jnp.int32),
        pltpu.VMEM((EPT,), jnp.int32),
        pltpu.VMEM((EPT,), jnp.int32),
        pltpu.VMEM((EPT,), jnp.int32),
        pltpu.VMEM((DCHUNK,), jnp.float32),
        pltpu.VMEM((HR,), jnp.float32),
    ]
    + [pltpu.SemaphoreType.DMA] * DBUF,
)
def _prep(src2_hbm, dst2_hbm, z1_hbm, ns_hbm, nd_hbm, lists_hbm,
          hist, idxa, srcp, dstp, l0s, l1s, l0d, l1d, onesb, normv, *sems):
    c = lax.axis_index("c")
    s = lax.axis_index("s")
    t = c * NS + s
    r0 = pl.multiple_of(s * HR, 128)
    pltpu.sync_copy(z1_hbm.at[pl.ds(r0, HR)], hist.at[pl.ds(r0, HR)])
    for j in range(DCHUNK // 16):
        onesb[pl.ds(j * 16, 16)] = jnp.full((16,), 1.0, jnp.float32)

    @pl.when(c == 0)
    def _():
        pltpu.sync_copy(src2_hbm.at[pl.ds(s * DITERS, DITERS)], idxa)

    @pl.when(c == 1)
    def _():
        pltpu.sync_copy(dst2_hbm.at[pl.ds(s * DITERS, DITERS)], idxa)

    pltpu.sync_copy(src2_hbm.at[pl.ds(t * PCH, PCH)], srcp)
    pltpu.sync_copy(dst2_hbm.at[pl.ds(t * PCH, PCH)], dstp)
    plsc.subcore_barrier()

    def _hstart(b, ci):
        pltpu.async_copy(onesb, hist.at[idxa.at[ci]], sems[b], add=True)

    def _hwait(b):
        pltpu.make_async_copy(onesb, hist.at[idxa.at[0]], sems[b]).wait()

    for b in range(DBUF):
        _hstart(b, b)

    def hbody(g, carry):
        for b in range(DBUF):
            _hwait(b)
            _hstart(b, (g + 1) * DBUF + b)
        return carry

    lax.fori_loop(0, DITERS // DBUF - 1, hbody, 0)
    for b in range(DBUF):
        _hwait(b)

    hh = jnp.full((16,), HHALF, jnp.int32)
    zero = jnp.zeros((16,), jnp.int32)
    padv = jnp.full((16,), PAD_NODE, jnp.int32)

    def prow(r, carry):
        for co in range(DCHUNK // 16):
            cs = pl.ds(co * 16, 16)
            fs = pl.ds(pl.multiple_of(r * DCHUNK + co * 16, 16), 16)
            sv = srcp[r, cs]
            dv = dstp[r, cs]
            m0 = sv < hh
            l0s[fs] = jnp.where(m0, sv, zero)
            l0d[fs] = jnp.where(m0, dv, padv)
            l1s[fs] = jnp.where(m0, zero, sv - hh)
            l1d[fs] = jnp.where(m0, padv, dv)
        return carry

    lax.fori_loop(0, PCH, prow, 0)
    pltpu.sync_copy(l0s, lists_hbm.at[pl.ds(((0 * NT + t) * 2 + 0) * EPT, EPT)])
    pltpu.sync_copy(l0d, lists_hbm.at[pl.ds(((0 * NT + t) * 2 + 1) * EPT, EPT)])
    pltpu.sync_copy(l1s, lists_hbm.at[pl.ds(((1 * NT + t) * 2 + 0) * EPT, EPT)])
    pltpu.sync_copy(l1d, lists_hbm.at[pl.ds(((1 * NT + t) * 2 + 1) * EPT, EPT)])

    plsc.subcore_barrier()
    pltpu.sync_copy(hist.at[pl.ds(r0, HR)], normv)
    for j in range(HR // 16):
        normv[pl.ds(j * 16, 16)] = _rsqrt16(normv[pl.ds(j * 16, 16)])

    @pl.when(c == 0)
    def _():
        pltpu.sync_copy(normv, ns_hbm.at[pl.ds(r0, HR)])

    @pl.when(c == 1)
    def _():
        pltpu.sync_copy(normv, nd_hbm.at[pl.ds(r0, HR)])


@functools.partial(
    pl.kernel,
    out_type=jax.ShapeDtypeStruct((NC, NPA, D), jnp.float32),
    mesh=_mesh,
    scratch_types=[
        pltpu.VMEM_SHARED((HHALF, D), jnp.float32),
        pltpu.VMEM_SHARED((NPA, D), jnp.float32),
        pltpu.VMEM((2 * NBUF, ECHUNK), jnp.int32),
        pltpu.VMEM((2 * NBUF, ECHUNK), jnp.int32),
    ]
    + [pltpu.VMEM((ECHUNK, D), jnp.float32)] * NBUF
    + [pltpu.SemaphoreType.DMA] * (2 * NBUF + 2),
)
def _edge_scatter(h_hbm, lists_hbm, z2_hbm, out_hbm, hsh, acc, sidxg, didxg, *bufs):
    rows = bufs[:NBUF]
    gsem = bufs[NBUF:2 * NBUF]
    ssem = bufs[2 * NBUF:3 * NBUF]
    issem, idsem = bufs[3 * NBUF:]
    c = lax.axis_index("c")
    s = lax.axis_index("s")
    r0 = pl.multiple_of(s * ACC_RPT, 8)
    h0 = pl.multiple_of(s * HROWS, 8)
    pltpu.sync_copy(z2_hbm.at[pl.ds(r0, ACC_RPT)], acc.at[pl.ds(r0, ACC_RPT)])
    pltpu.sync_copy(h_hbm.at[pl.ds(c * HHALF + h0, HROWS)], hsh.at[pl.ds(h0, HROWS)])
    plsc.subcore_barrier()

    def _gstart(b, slot):
        pltpu.async_copy(hsh.at[sidxg.at[slot * NBUF + b]], rows[b], gsem[b])

    def _gwait(b):
        pltpu.make_async_copy(hsh.at[sidxg.at[0]], rows[b], gsem[b]).wait()

    def _sstart(b, slot):
        pltpu.async_copy(rows[b], acc.at[didxg.at[slot * NBUF + b]], ssem[b], add=True)

    def _swait(b):
        pltpu.make_async_copy(rows[b], acc.at[didxg.at[0]], ssem[b]).wait()

    for j in range(2):
        t2 = 2 * s + j
        sl = lists_hbm.at[c, t2, 0]
        dl = lists_hbm.at[c, t2, 1]

        def _istart(g, slot, sl=sl, dl=dl):
            pltpu.async_copy(sl.at[pl.ds(g * NBUF, NBUF)],
                             sidxg.at[pl.ds(slot * NBUF, NBUF)], issem)
            pltpu.async_copy(dl.at[pl.ds(g * NBUF, NBUF)],
                             didxg.at[pl.ds(slot * NBUF, NBUF)], idsem)

        def _iwait(sl=sl, dl=dl):
            pltpu.make_async_copy(sl.at[pl.ds(0, NBUF)],
                                  sidxg.at[pl.ds(0, NBUF)], issem).wait()
            pltpu.make_async_copy(dl.at[pl.ds(0, NBUF)],
                                  didxg.at[pl.ds(0, NBUF)], idsem).wait()

        pltpu.sync_copy(sl.at[pl.ds(0, NBUF)], sidxg.at[pl.ds(0, NBUF)])
        pltpu.sync_copy(dl.at[pl.ds(0, NBUF)], didxg.at[pl.ds(0, NBUF)])
        for b in range(NBUF):
            _gstart(b, 0)

        def body(g, carry, _istart=_istart, _iwait=_iwait):
            p = lax.rem(g, 2)
            _istart(g + 1, 1 - p)
            for b in range(NBUF):
                _gwait(b)
                _sstart(b, p)
            _iwait()
            for b in range(NBUF):
                _swait(b)
                _gstart(b, 1 - p)
            return carry

        lax.fori_loop(0, LGRP - 1, body, 0)
        pl_last = (LGRP - 1) % 2
        for b in range(NBUF):
            _gwait(b)
            _sstart(b, pl_last)
        for b in range(NBUF):
            _swait(b)

    plsc.subcore_barrier()
    pltpu.sync_copy(acc.at[pl.ds(r0, ACC_RPT)], out_hbm.at[c, pl.ds(r0, ACC_RPT)])


def _dot(a, b):
    return lax.dot_general(a, b, (((1,), (0,)), ((), ())),
                           preferred_element_type=jnp.float32,
                           precision=lax.Precision.HIGHEST)


def _mm_body(x_ref, ns_ref, w_ref, o_ref):
    o_ref[...] = _dot(x_ref[...] * ns_ref[...], w_ref[...])


def _mm_call(x, ns2, W):
    return pl.pallas_call(
        _mm_body,
        grid=(GRID,),
        in_specs=[
            pl.BlockSpec((BLKH, D), lambda i: (i, 0)),
            pl.BlockSpec((BLKH, 1), lambda i: (i, 0)),
            pl.BlockSpec((D, D), lambda i: (0, 0)),
        ],
        out_specs=pl.BlockSpec((BLKH, D), lambda i: (i, 0)),
        out_shape=jax.ShapeDtypeStruct((NPH, D), jnp.float32),
    )(x, ns2, W)


def _mid_body(a_ref, nd_ref, ns_ref, b_ref, w_ref, o_ref):
    agg = a_ref[0] + a_ref[1]
    mid = jnp.maximum(agg * nd_ref[...] + b_ref[...], 0.0)
    o_ref[...] = _dot(mid * ns_ref[...], w_ref[...])


def _mid_call(aggp, nd2, ns2, b1r, W2):
    return pl.pallas_call(
        _mid_body,
        grid=(GRID,),
        in_specs=[
            pl.BlockSpec((NC, BLKA, D), lambda i: (0, i, 0)),
            pl.BlockSpec((BLKA, 1), lambda i: (i, 0)),
            pl.BlockSpec((BLKA, 1), lambda i: (i, 0)),
            pl.BlockSpec((1, D), lambda i: (0, 0)),
            pl.BlockSpec((D, D), lambda i: (0, 0)),
        ],
        out_specs=pl.BlockSpec((BLKA, D), lambda i: (i, 0)),
        out_shape=jax.ShapeDtypeStruct((NPA, D), jnp.float32),
    )(aggp, nd2, ns2, b1r, W2)


def _fin_body(a_ref, nd_ref, b_ref, o_ref):
    agg = a_ref[0] + a_ref[1]
    o_ref[...] = jnp.maximum(agg * nd_ref[...] + b_ref[...], 0.0)


def _fin_call(aggp, nd2, b2r):
    return pl.pallas_call(
        _fin_body,
        grid=(GRID,),
        in_specs=[
            pl.BlockSpec((NC, BLKA, D), lambda i: (0, i, 0)),
            pl.BlockSpec((BLKA, 1), lambda i: (i, 0)),
            pl.BlockSpec((1, D), lambda i: (0, 0)),
        ],
        out_specs=pl.BlockSpec((BLKA, D), lambda i: (i, 0)),
        out_shape=jax.ShapeDtypeStruct((NPA, D), jnp.float32),
    )(aggp, nd2, b2r)


def kernel(node_features, edge_index, W1, b1, W2, b2):
    src = edge_index[0].astype(jnp.int32).reshape(NT, E // NT)
    dst = edge_index[1].astype(jnp.int32).reshape(NT, E // NT)
    padw = ((0, 0), (0, EPT - E // NT))
    src_p = jnp.pad(src, padw, constant_values=PAD_NODE).reshape(EPAD // DCHUNK, DCHUNK)
    dst_p = jnp.pad(dst, padw, constant_values=PAD_NODE).reshape(EPAD // DCHUNK, DCHUNK)
    x_p = jnp.pad(node_features, ((0, NPH - N), (0, 0)))
    z1 = jnp.zeros((NHIST,), jnp.float32)
    z2 = jnp.zeros((NPA, D), jnp.float32)

    ns, nd, lists = _prep(src_p, dst_p, z1)
    lists_c = lists.reshape(NC, NT, 2, LCH, ECHUNK)
    ns2 = ns[:NPH, None]
    nsa = ns[:NPA, None]
    nda = nd[:NPA, None]

    h1 = _mm_call(x_p, ns2, W1)
    agg1 = _edge_scatter(h1, lists_c, z2)
    h2 = _mid_call(agg1, nda, nsa, b1[None, :], W2)
    h2p = jnp.pad(h2, ((0, NPH - NPA), (0, 0)))
    agg2 = _edge_scatter(h2p, lists_c, z2)
    out = _fin_call(agg2, nda, b2[None, :])
    return out[:N]


# speedup vs baseline: 4.6440x; 1.0036x over previous
import functools

import jax
import jax.numpy as jnp
from jax import lax
from jax.experimental import pallas as pl
from jax.experimental.pallas import tpu as pltpu
from jax.experimental.pallas import tpu_sc as plsc

N = 10000
NPH = 10240
HHALF = NPH // 2
NPA = 10112
NHIST = 10240
E = 320000
D = 128
NC = 2
NS = 16
NT = NC * NS
EPT = 10240
EPAD = NT * EPT
ACC_RPT = NPA // NS
HROWS = HHALF // NS
HR = NHIST // NS
PAD_NODE = 10016
ECHUNK = 32
NBUF = 2
LCH = EPT // ECHUNK
LGRP = LCH // NBUF
DCHUNK = 64
DITERS = EPAD // NS // DCHUNK
DBUF = 4
PCH = EPT // DCHUNK
BLKH = 640
BLKA = 632
GRID = 16

_mesh = plsc.VectorSubcoreMesh(core_axis_name="c", subcore_axis_name="s")


def _rsqrt16(d):
    i = lax.bitcast_convert_type(d, jnp.int32)
    i = jnp.int32(0x5F3759DF) - lax.shift_right_arithmetic(i, 1)
    y = lax.bitcast_convert_type(i, jnp.float32)
    for _ in range(3):
        y = y * (jnp.float32(1.5) - jnp.float32(0.5) * d * y * y)
    return jnp.where(d > jnp.float32(0.5), y, jnp.float32(0.0))


@functools.partial(
    pl.kernel,
    out_type=(
        jax.ShapeDtypeStruct((NHIST,), jnp.float32),
        jax.ShapeDtypeStruct((NHIST,), jnp.float32),
        jax.ShapeDtypeStruct((NC * NT * 2 * EPT,), jnp.int32),
    ),
    mesh=_mesh,
    scratch_types=[
        pltpu.VMEM_SHARED((NHIST,), jnp.float32),
        pltpu.VMEM((DITERS, DCHUNK), jnp.int32),
        pltpu.VMEM((PCH, DCHUNK), jnp.int32),
        pltpu.VMEM((PCH, DCHUNK), jnp.int32),
        pltpu.VMEM((EPT,), jnp.int32),
        pltpu.VMEM((EPT,), jnp.int32),
        pltpu.VMEM((EPT,), jnp.int32),
        pltpu.VMEM((EPT,), jnp.int32),
        pltpu.VMEM((DCHUNK,), jnp.float32),
        pltpu.VMEM((HR,), jnp.float32),
    ]
    + [pltpu.SemaphoreType.DMA] * DBUF,
)
def _prep(src2_hbm, dst2_hbm, z1_hbm, ns_hbm, nd_hbm, lists_hbm,
          hist, idxa, srcp, dstp, l0s, l1s, l0d, l1d, onesb, normv, *sems):
    c = lax.axis_index("c")
    s = lax.axis_index("s")
    t = c * NS + s
    r0 = pl.multiple_of(s * HR, 128)
    pltpu.sync_copy(z1_hbm.at[pl.ds(r0, HR)], hist.at[pl.ds(r0, HR)])
    for j in range(DCHUNK // 16):
        onesb[pl.ds(j * 16, 16)] = jnp.full((16,), 1.0, jnp.float32)

    @pl.when(c == 0)
    def _():
        pltpu.sync_copy(src2_hbm.at[pl.ds(s * DITERS, DITERS)], idxa)

    @pl.when(c == 1)
    def _():
        pltpu.sync_copy(dst2_hbm.at[pl.ds(s * DITERS, DITERS)], idxa)

    pltpu.sync_copy(src2_hbm.at[pl.ds(t * PCH, PCH)], srcp)
    pltpu.sync_copy(dst2_hbm.at[pl.ds(t * PCH, PCH)], dstp)
    plsc.subcore_barrier()

    def _hstart(b, ci):
        pltpu.async_copy(onesb, hist.at[idxa.at[ci]], sems[b], add=True)

    def _hwait(b):
        pltpu.make_async_copy(onesb, hist.at[idxa.at[0]], sems[b]).wait()

    for b in range(DBUF):
        _hstart(b, b)

    def hbody(g, carry):
        for b in range(DBUF):
            _hwait(b)
            _hstart(b, (g + 1) * DBUF + b)
        return carry

    lax.fori_loop(0, DITERS // DBUF - 1, hbody, 0)
    for b in range(DBUF):
        _hwait(b)

    hh = jnp.full((16,), HHALF, jnp.int32)
    zero = jnp.zeros((16,), jnp.int32)
    padv = jnp.full((16,), PAD_NODE, jnp.int32)

    def prow(r, carry):
        for co in range(DCHUNK // 16):
            cs = pl.ds(co * 16, 16)
            fs = pl.ds(pl.multiple_of(r * DCHUNK + co * 16, 16), 16)
            sv = srcp[r, cs]
            dv = dstp[r, cs]
            m0 = sv < hh
            l0s[fs] = jnp.where(m0, sv, zero)
            l0d[fs] = jnp.where(m0, dv, padv)
            l1s[fs] = jnp.where(m0, zero, sv - hh)
            l1d[fs] = jnp.where(m0, padv, dv)
        return carry

    lax.fori_loop(0, PCH, prow, 0)
    pltpu.sync_copy(l0s, lists_hbm.at[pl.ds(((0 * NT + t) * 2 + 0) * EPT, EPT)])
    pltpu.sync_copy(l0d, lists_hbm.at[pl.ds(((0 * NT + t) * 2 + 1) * EPT, EPT)])
    pltpu.sync_copy(l1s, lists_hbm.at[pl.ds(((1 * NT + t) * 2 + 0) * EPT, EPT)])
    pltpu.sync_copy(l1d, lists_hbm.at[pl.ds(((1 * NT + t) * 2 + 1) * EPT, EPT)])

    plsc.subcore_barrier()
    pltpu.sync_copy(hist.at[pl.ds(r0, HR)], normv)
    for j in range(HR // 16):
        normv[pl.ds(j * 16, 16)] = _rsqrt16(normv[pl.ds(j * 16, 16)])

    @pl.when(c == 0)
    def _():
        pltpu.sync_copy(normv, ns_hbm.at[pl.ds(r0, HR)])

    @pl.when(c == 1)
    def _():
        pltpu.sync_copy(normv, nd_hbm.at[pl.ds(r0, HR)])


@functools.partial(
    pl.kernel,
    out_type=jax.ShapeDtypeStruct((NC, NPA, D), jnp.float32),
    mesh=_mesh,
    scratch_types=[
        pltpu.VMEM_SHARED((HHALF, D), jnp.float32),
        pltpu.VMEM_SHARED((NPA, D), jnp.float32),
        pltpu.VMEM((2 * NBUF, ECHUNK), jnp.int32),
        pltpu.VMEM((2 * NBUF, ECHUNK), jnp.int32),
    ]
    + [pltpu.VMEM((ECHUNK, D), jnp.float32)] * NBUF
    + [pltpu.SemaphoreType.DMA] * (2 * NBUF + 2),
)
def _edge_scatter(h_hbm, lists_hbm, z2_hbm, out_hbm, hsh, acc, sidxg, didxg, *bufs):
    rows = bufs[:NBUF]
    gsem = bufs[NBUF:2 * NBUF]
    ssem = bufs[2 * NBUF:3 * NBUF]
    issem, idsem = bufs[3 * NBUF:]
    c = lax.axis_index("c")
    s = lax.axis_index("s")
    r0 = pl.multiple_of(s * ACC_RPT, 8)
    h0 = pl.multiple_of(s * HROWS, 8)
    pltpu.sync_copy(z2_hbm.at[pl.ds(r0, ACC_RPT)], acc.at[pl.ds(r0, ACC_RPT)])
    pltpu.sync_copy(h_hbm.at[pl.ds(c * HHALF + h0, HROWS)], hsh.at[pl.ds(h0, HROWS)])
    plsc.subcore_barrier()

    def _gstart(b, slot):
        pltpu.async_copy(hsh.at[sidxg.at[slot * NBUF + b]], rows[b], gsem[b])

    def _gwait(b):
        pltpu.make_async_copy(hsh.at[sidxg.at[0]], rows[b], gsem[b]).wait()

    def _sstart(b, slot):
        pltpu.async_copy(rows[b], acc.at[didxg.at[slot * NBUF + b]], ssem[b], add=True)

    def _swait(b):
        pltpu.make_async_copy(rows[b], acc.at[didxg.at[0]], ssem[b]).wait()

    for j in range(2):
        t2 = 2 * s + j
        sl = lists_hbm.at[c, t2, 0]
        dl = lists_hbm.at[c, t2, 1]

        def _istart(g, slot, sl=sl, dl=dl):
            pltpu.async_copy(sl.at[pl.ds(g * NBUF, NBUF)],
                             sidxg.at[pl.ds(slot * NBUF, NBUF)], issem)
            pltpu.async_copy(dl.at[pl.ds(g * NBUF, NBUF)],
                             didxg.at[pl.ds(slot * NBUF, NBUF)], idsem)

        def _iwait(sl=sl, dl=dl):
            pltpu.make_async_copy(sl.at[pl.ds(0, NBUF)],
                                  sidxg.at[pl.ds(0, NBUF)], issem).wait()
            pltpu.make_async_copy(dl.at[pl.ds(0, NBUF)],
                                  didxg.at[pl.ds(0, NBUF)], idsem).wait()

        pltpu.sync_copy(sl.at[pl.ds(0, NBUF)], sidxg.at[pl.ds(0, NBUF)])
        pltpu.sync_copy(dl.at[pl.ds(0, NBUF)], didxg.at[pl.ds(0, NBUF)])
        for b in range(NBUF):
            _gstart(b, 0)

        def body(g, carry, _istart=_istart, _iwait=_iwait):
            p = lax.rem(g, 2)
            _istart(g + 1, 1 - p)
            for b in range(NBUF):
                _gwait(b)
                _sstart(b, p)
            _iwait()
            for b in range(NBUF):
                _swait(b)
                _gstart(b, 1 - p)
            return carry

        lax.fori_loop(0, LGRP - 1, body, 0)
        pl_last = (LGRP - 1) % 2
        for b in range(NBUF):
            _gwait(b)
            _sstart(b, pl_last)
        for b in range(NBUF):
            _swait(b)

    plsc.subcore_barrier()
    pltpu.sync_copy(acc.at[pl.ds(r0, ACC_RPT)], out_hbm.at[c, pl.ds(r0, ACC_RPT)])


def _dot(a, b):
    return lax.dot_general(a, b, (((1,), (0,)), ((), ())),
                           preferred_element_type=jnp.float32,
                           precision=lax.Precision.HIGHEST)


def _mm_body(x_ref, ns_ref, w_ref, o_ref):
    o_ref[...] = _dot(x_ref[...] * ns_ref[...], w_ref[...])


def _mm_call(x, ns2, W):
    return pl.pallas_call(
        _mm_body,
        grid=(GRID,),
        in_specs=[
            pl.BlockSpec((BLKH, D), lambda i: (i, 0)),
            pl.BlockSpec((BLKH, 1), lambda i: (i, 0)),
            pl.BlockSpec((D, D), lambda i: (0, 0)),
        ],
        out_specs=pl.BlockSpec((BLKH, D), lambda i: (i, 0)),
        out_shape=jax.ShapeDtypeStruct((NPH, D), jnp.float32),
    )(x, ns2, W)


def _mid_body(a_ref, nd_ref, ns_ref, b_ref, w_ref, o_ref):
    agg = a_ref[0] + a_ref[1]
    mid = jnp.maximum(agg * nd_ref[...] + b_ref[...], 0.0)
    o_ref[...] = _dot(mid * ns_ref[...], w_ref[...])


def _mid_call(aggp, nd2, ns2, b1r, W2):
    return pl.pallas_call(
        _mid_body,
        grid=(GRID,),
        in_specs=[
            pl.BlockSpec((NC, BLKA, D), lambda i: (0, i, 0)),
            pl.BlockSpec((BLKA, 1), lambda i: (i, 0)),
            pl.BlockSpec((BLKA, 1), lambda i: (i, 0)),
            pl.BlockSpec((1, D), lambda i: (0, 0)),
            pl.BlockSpec((D, D), lambda i: (0, 0)),
        ],
        out_specs=pl.BlockSpec((BLKA, D), lambda i: (i, 0)),
        out_shape=jax.ShapeDtypeStruct((NPH, D), jnp.float32),
    )(aggp, nd2, ns2, b1r, W2)


def _fin_body(a_ref, nd_ref, b_ref, o_ref):
    agg = a_ref[0] + a_ref[1]
    o_ref[...] = jnp.maximum(agg * nd_ref[...] + b_ref[...], 0.0)


def _fin_call(aggp, nd2, b2r):
    return pl.pallas_call(
        _fin_body,
        grid=(GRID,),
        in_specs=[
            pl.BlockSpec((NC, BLKA, D), lambda i: (0, i, 0)),
            pl.BlockSpec((BLKA, 1), lambda i: (i, 0)),
            pl.BlockSpec((1, D), lambda i: (0, 0)),
        ],
        out_specs=pl.BlockSpec((BLKA, D), lambda i: (i, 0)),
        out_shape=jax.ShapeDtypeStruct((NPA, D), jnp.float32),
    )(aggp, nd2, b2r)


def kernel(node_features, edge_index, W1, b1, W2, b2):
    src = edge_index[0].astype(jnp.int32).reshape(NT, E // NT)
    dst = edge_index[1].astype(jnp.int32).reshape(NT, E // NT)
    padw = ((0, 0), (0, EPT - E // NT))
    src_p = jnp.pad(src, padw, constant_values=PAD_NODE).reshape(EPAD // DCHUNK, DCHUNK)
    dst_p = jnp.pad(dst, padw, constant_values=PAD_NODE).reshape(EPAD // DCHUNK, DCHUNK)
    x_p = jnp.pad(node_features, ((0, NPH - N), (0, 0)))
    z1 = jnp.zeros((NHIST,), jnp.float32)
    z2 = jnp.zeros((NPA, D), jnp.float32)

    ns, nd, lists = _prep(src_p, dst_p, z1)
    lists_c = lists.reshape(NC, NT, 2, LCH, ECHUNK)
    ns2 = ns[:NPH, None]
    nsa = ns[:NPA, None]
    nda = nd[:NPA, None]

    h1 = _mm_call(x_p, ns2, W1)
    agg1 = _edge_scatter(h1, lists_c, z2)
    h2 = _mid_call(agg1, nda, nsa, b1[None, :], W2)
    agg2 = _edge_scatter(h2, lists_c, z2)
    out = _fin_call(agg2, nda, b2[None, :])
    return out[:N]


# speedup vs baseline: 4.8544x; 1.0453x over previous
import functools

import jax
import jax.numpy as jnp
from jax import lax
from jax.experimental import pallas as pl
from jax.experimental.pallas import tpu as pltpu
from jax.experimental.pallas import tpu_sc as plsc

N = 10000
NPH = 10240
HHALF = NPH // 2
NPA = 10112
NHIST = 10240
E = 320000
D = 128
NC = 2
NS = 16
NT = NC * NS
EPT = 10240
EPAD = NT * EPT
ACC_RPT = NPA // NS
HROWS = HHALF // NS
HR = NHIST // NS
PAD_NODE = 10016
ECHUNK = 32
NBUF = 2
LCH = EPT // ECHUNK
LGRP = LCH // NBUF
DCHUNK = 64
DITERS = EPAD // NS // DCHUNK
DBUF = 4
PCH = EPT // DCHUNK
BLKH = 640
BLKA = 632
GRID = 16

_mesh = plsc.VectorSubcoreMesh(core_axis_name="c", subcore_axis_name="s")


def _rsqrt16(d):
    i = lax.bitcast_convert_type(d, jnp.int32)
    i = jnp.int32(0x5F3759DF) - lax.shift_right_arithmetic(i, 1)
    y = lax.bitcast_convert_type(i, jnp.float32)
    for _ in range(3):
        y = y * (jnp.float32(1.5) - jnp.float32(0.5) * d * y * y)
    return jnp.where(d > jnp.float32(0.5), y, jnp.float32(0.0))


@functools.partial(
    pl.kernel,
    out_type=(
        jax.ShapeDtypeStruct((NHIST,), jnp.float32),
        jax.ShapeDtypeStruct((NHIST,), jnp.float32),
        jax.ShapeDtypeStruct((NC * NT * 2 * EPT,), jnp.int32),
    ),
    mesh=_mesh,
    scratch_types=[
        pltpu.VMEM_SHARED((NHIST,), jnp.float32),
        pltpu.VMEM((DITERS, DCHUNK), jnp.int32),
        pltpu.VMEM((PCH, DCHUNK), jnp.int32),
        pltpu.VMEM((PCH, DCHUNK), jnp.int32),
        pltpu.VMEM((EPT,), jnp.int32),
        pltpu.VMEM((EPT,), jnp.int32),
        pltpu.VMEM((EPT,), jnp.int32),
        pltpu.VMEM((EPT,), jnp.int32),
        pltpu.VMEM((DCHUNK,), jnp.float32),
        pltpu.VMEM((HR,), jnp.float32),
    ]
    + [pltpu.SemaphoreType.DMA] * DBUF,
)
def _prep(src2_hbm, dst2_hbm, z1_hbm, ns_hbm, nd_hbm, lists_hbm,
          hist, idxa, srcp, dstp, l0s, l1s, l0d, l1d, onesb, normv, *sems):
    c = lax.axis_index("c")
    s = lax.axis_index("s")
    t = c * NS + s
    r0 = pl.multiple_of(s * HR, 128)
    pltpu.sync_copy(z1_hbm.at[pl.ds(r0, HR)], hist.at[pl.ds(r0, HR)])
    for j in range(DCHUNK // 16):
        onesb[pl.ds(j * 16, 16)] = jnp.full((16,), 1.0, jnp.float32)

    @pl.when(c == 0)
    def _():
        pltpu.sync_copy(src2_hbm.at[pl.ds(s * DITERS, DITERS)], idxa)

    @pl.when(c == 1)
    def _():
        pltpu.sync_copy(dst2_hbm.at[pl.ds(s * DITERS, DITERS)], idxa)

    pltpu.sync_copy(src2_hbm.at[pl.ds(t * PCH, PCH)], srcp)
    pltpu.sync_copy(dst2_hbm.at[pl.ds(t * PCH, PCH)], dstp)
    plsc.subcore_barrier()

    def _hstart(b, ci):
        pltpu.async_copy(onesb, hist.at[idxa.at[ci]], sems[b], add=True)

    def _hwait(b):
        pltpu.make_async_copy(onesb, hist.at[idxa.at[0]], sems[b]).wait()

    for b in range(DBUF):
        _hstart(b, b)

    def hbody(g, carry):
        for b in range(DBUF):
            _hwait(b)
            _hstart(b, (g + 1) * DBUF + b)
        return carry

    lax.fori_loop(0, DITERS // DBUF - 1, hbody, 0)
    for b in range(DBUF):
        _hwait(b)

    hh = jnp.full((16,), HHALF, jnp.int32)
    lane = lax.iota(jnp.int32, 16)

    def prow(r, carry):
        rv = lax.broadcast_in_dim(r * 16, (16,), ()) + lane
        jsrc = jnp.bitwise_and(rv, jnp.full((16,), 4095, jnp.int32))
        jdst = jnp.full((16,), 10000, jnp.int32) + jnp.bitwise_and(
            rv, jnp.full((16,), 63, jnp.int32))
        for co in range(DCHUNK // 16):
            cs = pl.ds(co * 16, 16)
            fs = pl.ds(pl.multiple_of(r * DCHUNK + co * 16, 16), 16)
            sv = srcp[r, cs]
            dv = dstp[r, cs]
            m0 = sv < hh
            l0s[fs] = jnp.where(m0, sv, jsrc)
            l0d[fs] = jnp.where(m0, dv, jdst)
            l1s[fs] = jnp.where(m0, jsrc, sv - hh)
            l1d[fs] = jnp.where(m0, jdst, dv)
        return carry

    lax.fori_loop(0, PCH, prow, 0)
    pltpu.sync_copy(l0s, lists_hbm.at[pl.ds(((0 * NT + t) * 2 + 0) * EPT, EPT)])
    pltpu.sync_copy(l0d, lists_hbm.at[pl.ds(((0 * NT + t) * 2 + 1) * EPT, EPT)])
    pltpu.sync_copy(l1s, lists_hbm.at[pl.ds(((1 * NT + t) * 2 + 0) * EPT, EPT)])
    pltpu.sync_copy(l1d, lists_hbm.at[pl.ds(((1 * NT + t) * 2 + 1) * EPT, EPT)])

    plsc.subcore_barrier()
    pltpu.sync_copy(hist.at[pl.ds(r0, HR)], normv)
    for j in range(HR // 16):
        normv[pl.ds(j * 16, 16)] = _rsqrt16(normv[pl.ds(j * 16, 16)])

    @pl.when(c == 0)
    def _():
        pltpu.sync_copy(normv, ns_hbm.at[pl.ds(r0, HR)])

    @pl.when(c == 1)
    def _():
        pltpu.sync_copy(normv, nd_hbm.at[pl.ds(r0, HR)])


@functools.partial(
    pl.kernel,
    out_type=jax.ShapeDtypeStruct((NC, NPA, D), jnp.float32),
    mesh=_mesh,
    scratch_types=[
        pltpu.VMEM_SHARED((HHALF, D), jnp.float32),
        pltpu.VMEM_SHARED((NPA, D), jnp.float32),
        pltpu.VMEM((2 * NBUF, ECHUNK), jnp.int32),
        pltpu.VMEM((2 * NBUF, ECHUNK), jnp.int32),
    ]
    + [pltpu.VMEM((ECHUNK, D), jnp.float32)] * NBUF
    + [pltpu.SemaphoreType.DMA] * (2 * NBUF + 2),
)
def _edge_scatter(h_hbm, lists_hbm, z2_hbm, out_hbm, hsh, acc, sidxg, didxg, *bufs):
    rows = bufs[:NBUF]
    gsem = bufs[NBUF:2 * NBUF]
    ssem = bufs[2 * NBUF:3 * NBUF]
    issem, idsem = bufs[3 * NBUF:]
    c = lax.axis_index("c")
    s = lax.axis_index("s")
    r0 = pl.multiple_of(s * ACC_RPT, 8)
    h0 = pl.multiple_of(s * HROWS, 8)
    pltpu.sync_copy(z2_hbm.at[pl.ds(r0, ACC_RPT)], acc.at[pl.ds(r0, ACC_RPT)])
    pltpu.sync_copy(h_hbm.at[pl.ds(c * HHALF + h0, HROWS)], hsh.at[pl.ds(h0, HROWS)])
    plsc.subcore_barrier()

    def _gstart(b, slot):
        pltpu.async_copy(hsh.at[sidxg.at[slot * NBUF + b]], rows[b], gsem[b])

    def _gwait(b):
        pltpu.make_async_copy(hsh.at[sidxg.at[0]], rows[b], gsem[b]).wait()

    def _sstart(b, slot):
        pltpu.async_copy(rows[b], acc.at[didxg.at[slot * NBUF + b]], ssem[b], add=True)

    def _swait(b):
        pltpu.make_async_copy(rows[b], acc.at[didxg.at[0]], ssem[b]).wait()

    for j in range(2):
        t2 = 2 * s + j
        sl = lists_hbm.at[c, t2, 0]
        dl = lists_hbm.at[c, t2, 1]

        def _istart(g, slot, sl=sl, dl=dl):
            pltpu.async_copy(sl.at[pl.ds(g * NBUF, NBUF)],
                             sidxg.at[pl.ds(slot * NBUF, NBUF)], issem)
            pltpu.async_copy(dl.at[pl.ds(g * NBUF, NBUF)],
                             didxg.at[pl.ds(slot * NBUF, NBUF)], idsem)

        def _iwait(sl=sl, dl=dl):
            pltpu.make_async_copy(sl.at[pl.ds(0, NBUF)],
                                  sidxg.at[pl.ds(0, NBUF)], issem).wait()
            pltpu.make_async_copy(dl.at[pl.ds(0, NBUF)],
                                  didxg.at[pl.ds(0, NBUF)], idsem).wait()

        pltpu.sync_copy(sl.at[pl.ds(0, NBUF)], sidxg.at[pl.ds(0, NBUF)])
        pltpu.sync_copy(dl.at[pl.ds(0, NBUF)], didxg.at[pl.ds(0, NBUF)])
        for b in range(NBUF):
            _gstart(b, 0)

        def body(g, carry, _istart=_istart, _iwait=_iwait):
            p = lax.rem(g, 2)
            _istart(g + 1, 1 - p)
            for b in range(NBUF):
                _gwait(b)
                _sstart(b, p)
            _iwait()
            for b in range(NBUF):
                _swait(b)
                _gstart(b, 1 - p)
            return carry

        lax.fori_loop(0, LGRP - 1, body, 0)
        pl_last = (LGRP - 1) % 2
        for b in range(NBUF):
            _gwait(b)
            _sstart(b, pl_last)
        for b in range(NBUF):
            _swait(b)

    plsc.subcore_barrier()
    pltpu.sync_copy(acc.at[pl.ds(r0, ACC_RPT)], out_hbm.at[c, pl.ds(r0, ACC_RPT)])


def _dot(a, b):
    return lax.dot_general(a, b, (((1,), (0,)), ((), ())),
                           preferred_element_type=jnp.float32,
                           precision=lax.Precision.HIGHEST)


def _mm_body(x_ref, ns_ref, w_ref, o_ref):
    o_ref[...] = _dot(x_ref[...] * ns_ref[...], w_ref[...])


def _mm_call(x, ns2, W):
    return pl.pallas_call(
        _mm_body,
        grid=(GRID,),
        in_specs=[
            pl.BlockSpec((BLKH, D), lambda i: (i, 0)),
            pl.BlockSpec((BLKH, 1), lambda i: (i, 0)),
            pl.BlockSpec((D, D), lambda i: (0, 0)),
        ],
        out_specs=pl.BlockSpec((BLKH, D), lambda i: (i, 0)),
        out_shape=jax.ShapeDtypeStruct((NPH, D), jnp.float32),
    )(x, ns2, W)


def _mid_body(a_ref, nd_ref, ns_ref, b_ref, w_ref, o_ref):
    agg = a_ref[0] + a_ref[1]
    mid = jnp.maximum(agg * nd_ref[...] + b_ref[...], 0.0)
    o_ref[...] = _dot(mid * ns_ref[...], w_ref[...])


def _mid_call(aggp, nd2, ns2, b1r, W2):
    return pl.pallas_call(
        _mid_body,
        grid=(GRID,),
        in_specs=[
            pl.BlockSpec((NC, BLKA, D), lambda i: (0, i, 0)),
            pl.BlockSpec((BLKA, 1), lambda i: (i, 0)),
            pl.BlockSpec((BLKA, 1), lambda i: (i, 0)),
            pl.BlockSpec((1, D), lambda i: (0, 0)),
            pl.BlockSpec((D, D), lambda i: (0, 0)),
        ],
        out_specs=pl.BlockSpec((BLKA, D), lambda i: (i, 0)),
        out_shape=jax.ShapeDtypeStruct((NPH, D), jnp.float32),
    )(aggp, nd2, ns2, b1r, W2)


def _fin_body(a_ref, nd_ref, b_ref, o_ref):
    agg = a_ref[0] + a_ref[1]
    o_ref[...] = jnp.maximum(agg * nd_ref[...] + b_ref[...], 0.0)


def _fin_call(aggp, nd2, b2r):
    return pl.pallas_call(
        _fin_body,
        grid=(GRID,),
        in_specs=[
            pl.BlockSpec((NC, BLKA, D), lambda i: (0, i, 0)),
            pl.BlockSpec((BLKA, 1), lambda i: (i, 0)),
            pl.BlockSpec((1, D), lambda i: (0, 0)),
        ],
        out_specs=pl.BlockSpec((BLKA, D), lambda i: (i, 0)),
        out_shape=jax.ShapeDtypeStruct((NPA, D), jnp.float32),
    )(aggp, nd2, b2r)


def kernel(node_features, edge_index, W1, b1, W2, b2):
    src = edge_index[0].astype(jnp.int32).reshape(NT, E // NT)
    dst = edge_index[1].astype(jnp.int32).reshape(NT, E // NT)
    padw = ((0, 0), (0, EPT - E // NT))
    src_p = jnp.pad(src, padw, constant_values=PAD_NODE).reshape(EPAD // DCHUNK, DCHUNK)
    dst_p = jnp.pad(dst, padw, constant_values=PAD_NODE).reshape(EPAD // DCHUNK, DCHUNK)
    x_p = jnp.pad(node_features, ((0, NPH - N), (0, 0)))
    z1 = jnp.zeros((NHIST,), jnp.float32)
    z2 = jnp.zeros((NPA, D), jnp.float32)

    ns, nd, lists = _prep(src_p, dst_p, z1)
    lists_c = lists.reshape(NC, NT, 2, LCH, ECHUNK)
    ns2 = ns[:NPH, None]
    nsa = ns[:NPA, None]
    nda = nd[:NPA, None]

    h1 = _mm_call(x_p, ns2, W1)
    agg1 = _edge_scatter(h1, lists_c, z2)
    h2 = _mid_call(agg1, nda, nsa, b1[None, :], W2)
    agg2 = _edge_scatter(h2, lists_c, z2)
    out = _fin_call(agg2, nda, b2[None, :])
    return out[:N]
